# Initial kernel scaffold; baseline (speedup 1.0000x reference)
import jax
import jax.numpy as jnp
from jax.experimental import pallas as pl


def kernel(img_features, inputs):
    raise NotImplementedError("write your pallas kernel here")


# speedup vs baseline: 1.9026x; 1.9026x over previous
import functools

import jax
import jax.numpy as jnp
from jax import lax
from jax.experimental import pallas as pl
from jax.experimental.pallas import tpu as pltpu
from jax.experimental.pallas import tpu_sc as plsc

H = W = 56
HW = H * W
D = 4 * 256
B = 8
P = 4096
NPTS = B * P
NW = 32
CHUNK = NPTS // NW
SUB = 16
NSUB = CHUNK // SUB
LANES = 16

SCALE_W = -248.0 / 111.5
SCALE_H = 248.0 / 111.5


@functools.partial(
    pl.kernel,
    mesh=plsc.VectorSubcoreMesh(core_axis_name="c", subcore_axis_name="s"),
    out_type=jax.ShapeDtypeStruct((NPTS, D), jnp.float32),
    scratch_types=[
        pltpu.VMEM((CHUNK,), jnp.float32),
        pltpu.VMEM((CHUNK,), jnp.float32),
        pltpu.VMEM((CHUNK,), jnp.float32),
        pltpu.VMEM((4, CHUNK), jnp.int32),
        pltpu.VMEM((4, CHUNK), jnp.float32),
        pltpu.VMEM((4, SUB, D), jnp.float32),
        pltpu.VMEM((SUB, D), jnp.float32),
        pltpu.SemaphoreType.DMA,
    ],
)
def _gproj_sc(table, xs_hbm, ys_hbm, zs_hbm, out_hbm,
              xs, ys, zs, idx, wgt, rows, outbuf, sem):
    wid = lax.axis_index("s") * 2 + lax.axis_index("c")
    base = wid * CHUNK
    rowbase = (base // P) * HW

    pltpu.sync_copy(xs_hbm.at[pl.ds(base, CHUNK)], xs)
    pltpu.sync_copy(ys_hbm.at[pl.ds(base, CHUNK)], ys)
    pltpu.sync_copy(zs_hbm.at[pl.ds(base, CHUNK)], zs)

    def compute_vec(i, _):
        sl = pl.ds(i * LANES, LANES)
        x = xs[sl]
        y = ys[sl]
        z = zs[sl] + (-0.8)
        w = jnp.clip((x / z) * SCALE_W, -1.0, 1.0)
        h = jnp.clip((y / z) * SCALE_H, -1.0, 1.0)
        ix = w * 28.0 + 27.5
        iy = h * 28.0 + 27.5
        tx = ix.astype(jnp.int32)
        ty = iy.astype(jnp.int32)
        ix0 = jnp.where(ix < tx.astype(jnp.float32), tx - 1, tx)
        iy0 = jnp.where(iy < ty.astype(jnp.float32), ty - 1, ty)
        fx1 = ix - ix0.astype(jnp.float32)
        fy1 = iy - iy0.astype(jnp.float32)
        fx0 = 1.0 - fx1
        fy0 = 1.0 - fy1
        wx0 = jnp.where(ix0 >= 0, fx0, 0.0)
        wx1 = jnp.where(ix0 < W - 1, fx1, 0.0)
        wy0 = jnp.where(iy0 >= 0, fy0, 0.0)
        wy1 = jnp.where(iy0 < H - 1, fy1, 0.0)
        cx0 = jnp.maximum(ix0, 0)
        cx1 = jnp.minimum(ix0 + 1, W - 1)
        cy0 = jnp.maximum(iy0, 0)
        cy1 = jnp.minimum(iy0 + 1, H - 1)
        r0 = rowbase + cy0 * W
        r1 = rowbase + cy1 * W
        idx[0, sl] = r0 + cx0
        idx[1, sl] = r0 + cx1
        idx[2, sl] = r1 + cx0
        idx[3, sl] = r1 + cx1
        wgt[0, sl] = wx0 * wy0
        wgt[1, sl] = wx1 * wy0
        wgt[2, sl] = wx0 * wy1
        wgt[3, sl] = wx1 * wy1
        return 0

    lax.fori_loop(0, CHUNK // LANES, compute_vec, 0)

    def sub(s, _):
        isl = pl.ds(s * SUB, SUB)
        cps = [pltpu.async_copy(table.at[idx.at[c, isl]], rows.at[c], sem)
               for c in range(4)]
        for cp in cps:
            cp.wait()
        wv = [wgt[c, isl] for c in range(4)]
        for p in range(SUB):
            w0 = wv[0][p]
            w1 = wv[1][p]
            w2 = wv[2][p]
            w3 = wv[3][p]

            def col(j, _):
                csl = pl.ds(j * LANES, LANES)
                acc = (rows[0, p, csl] * w0 + rows[1, p, csl] * w1
                       + rows[2, p, csl] * w2 + rows[3, p, csl] * w3)
                outbuf[p, csl] = acc
                return 0

            lax.fori_loop(0, D // LANES, col, 0)
        pltpu.sync_copy(outbuf, out_hbm.at[pl.ds(base + s * SUB, SUB)])
        return 0

    lax.fori_loop(0, NSUB, sub, 0)


def kernel(img_features, inputs):
    table = jnp.transpose(img_features, (1, 3, 4, 0, 2)).reshape(B * HW, D)
    coords = inputs.reshape(NPTS, 3)
    feats = _gproj_sc(table, coords[:, 0], coords[:, 1], coords[:, 2])
    return jnp.concatenate([inputs, feats.reshape(B, P, D)], axis=2)
